# baseline (speedup 1.0000x reference)
import functools

import jax
import jax.numpy as jnp
from jax import lax
from jax.experimental import pallas as pl
from jax.experimental.pallas import tpu as pltpu
from jax.experimental.pallas import tpu_sc as plsc

_NUM_COLORS = 256
_B, _H, _W = 16, 512, 512

_NC = 2
_NS = 16
_NW = _NC * _NS
_LANES = 16

_ROWS_PER_W = _H // 2
_CROWS = 8
_CHUNK = _CROWS * _W
_NCHUNK = _ROWS_PER_W // _CROWS
_NVEC = _CHUNK // _LANES
_VPR = _W // _LANES
_UNROLL = 8


def _make_sc_call():
    mesh = plsc.VectorSubcoreMesh(core_axis_name="c", subcore_axis_name="s")

    chunk_i32 = pltpu.VMEM((_CROWS, _W), jnp.int32)
    chunk3_f32 = pltpu.VMEM((3, _CROWS, _W), jnp.float32)

    @functools.partial(
        pl.kernel,
        mesh=mesh,
        out_type=jax.ShapeDtypeStruct((_B, 3, _H, _W), jnp.float32),
        scratch_types=[
            pltpu.VMEM((3, _NUM_COLORS), jnp.float32),
            pltpu.VMEM((3 * _NUM_COLORS,), jnp.float32),
            chunk_i32, chunk_i32,
            chunk3_f32, chunk3_f32,
            pltpu.SemaphoreType.DMA,
            pltpu.SemaphoreType.DMA,
            pltpu.SemaphoreType.DMA,
        ],
        compiler_params=pltpu.CompilerParams(needs_layout_passes=False),
    )
    def sc_kernel(x_hbm, tbl_hbm, out_hbm, tbl2d, tbl, inb0, inb1,
                  ob0, ob1, insem, osem0, osem1):
        inbufs = (inb0, inb1)
        outbufs = (ob0, ob1)
        osems = (osem0, osem1)
        wid = lax.axis_index("s") * _NC + lax.axis_index("c")
        img = wid // 2
        half = wid % 2
        row_base = half * _ROWS_PER_W

        def copy_in(k, slot):
            return pltpu.async_copy(
                x_hbm.at[img, 0, pl.ds(row_base + k * _CROWS, _CROWS), :],
                inbufs[slot],
                insem,
            )

        def copy_out(k, slot):
            return [
                pltpu.async_copy(
                    outbufs[slot],
                    out_hbm.at[img, :, pl.ds(row_base + k * _CROWS, _CROWS), :],
                    osems[slot],
                )
            ]

        def compute(slot):
            inb = inbufs[slot]
            outb = outbufs[slot]

            @plsc.parallel_loop(0, _NVEC, 1, unroll=_UNROLL)
            def _body(i):
                r = i // _VPR
                col = (i % _VPR) * _LANES
                v = inb[r, pl.ds(col, _LANES)]
                v = jnp.minimum(jnp.maximum(v, 0), _NUM_COLORS - 1)
                for c in range(3):
                    outb[c, r, pl.ds(col, _LANES)] = plsc.load_gather(
                        tbl, [v + (c * _NUM_COLORS)]
                    )

        in_handles = [None, None]
        out_handles = [None, None]
        in_handles[0] = copy_in(0, 0)
        pltpu.sync_copy(tbl_hbm, tbl2d)
        for c in range(3):
            for j in range(_NUM_COLORS // _LANES):
                tbl[pl.ds(c * _NUM_COLORS + j * _LANES, _LANES)] = (
                    tbl2d[c, pl.ds(j * _LANES, _LANES)]
                )
        for k in range(_NCHUNK):
            slot = k % 2
            nxt = (k + 1) % 2
            if k + 1 < _NCHUNK:
                in_handles[nxt] = copy_in(k + 1, nxt)
            in_handles[slot].wait()
            if out_handles[slot] is not None:
                for h in out_handles[slot]:
                    h.wait()
            compute(slot)
            out_handles[slot] = copy_out(k, slot)
        for slot in range(2):
            for h in out_handles[slot]:
                h.wait()

    return sc_kernel


_SC_CALL = _make_sc_call()


@jax.jit
def kernel(input_tensor, colors):
    return _SC_CALL(input_tensor, colors)


# speedup vs baseline: 1.2919x; 1.2919x over previous
import functools

import jax
import jax.numpy as jnp
from jax import lax
from jax.experimental import pallas as pl
from jax.experimental.pallas import tpu as pltpu
from jax.experimental.pallas import tpu_sc as plsc

_NUM_COLORS = 256
_B, _H, _W = 16, 512, 512

_NC = 2
_NS = 16
_NW = _NC * _NS
_LANES = 16

_ROWS_PER_W = _H // 2
_CROWS = 16
_CHUNK = _CROWS * _W
_NCHUNK = _ROWS_PER_W // _CROWS
_NVEC = _CHUNK // _LANES
_VPR = _W // _LANES
_UNROLL = 8


def _make_sc_call():
    mesh = plsc.VectorSubcoreMesh(core_axis_name="c", subcore_axis_name="s")

    chunk_i32 = pltpu.VMEM((_CROWS, _W), jnp.int32)
    chunk3_f32 = pltpu.VMEM((3, _CROWS, _W), jnp.float32)

    @functools.partial(
        pl.kernel,
        mesh=mesh,
        out_type=jax.ShapeDtypeStruct((_B, 3, _H, _W), jnp.float32),
        scratch_types=[
            pltpu.VMEM((3, _NUM_COLORS), jnp.float32),
            pltpu.VMEM((3 * _NUM_COLORS,), jnp.float32),
            chunk_i32, chunk_i32,
            chunk3_f32, chunk3_f32,
            pltpu.SemaphoreType.DMA,
            pltpu.SemaphoreType.DMA,
            pltpu.SemaphoreType.DMA,
        ],
        compiler_params=pltpu.CompilerParams(needs_layout_passes=False),
    )
    def sc_kernel(x_hbm, tbl_hbm, out_hbm, tbl2d, tbl, inb0, inb1,
                  ob0, ob1, insem, osem0, osem1):
        inbufs = (inb0, inb1)
        outbufs = (ob0, ob1)
        osems = (osem0, osem1)
        wid = lax.axis_index("s") * _NC + lax.axis_index("c")
        img = wid // 2
        half = wid % 2
        row_base = half * _ROWS_PER_W

        def in_src(k):
            return x_hbm.at[img, 0, pl.ds(row_base + k * _CROWS, _CROWS), :]

        def out_dst(k):
            return out_hbm.at[img, :, pl.ds(row_base + k * _CROWS, _CROWS), :]

        def compute(slot):
            inb = inbufs[slot]
            outb = outbufs[slot]

            @plsc.parallel_loop(0, _NVEC, 1, unroll=_UNROLL)
            def _body(i):
                r = i // _VPR
                col = (i % _VPR) * _LANES
                v = inb[r, pl.ds(col, _LANES)]
                v = jnp.minimum(jnp.maximum(v, 0), _NUM_COLORS - 1)
                for c in range(3):
                    outb[c, r, pl.ds(col, _LANES)] = plsc.load_gather(
                        tbl, [v + (c * _NUM_COLORS)]
                    )

        pltpu.async_copy(in_src(0), inbufs[0], insem)
        pltpu.async_copy(in_src(1), inbufs[1], insem)
        pltpu.sync_copy(tbl_hbm, tbl2d)
        for c in range(3):
            for j in range(_NUM_COLORS // _LANES):
                tbl[pl.ds(c * _NUM_COLORS + j * _LANES, _LANES)] = (
                    tbl2d[c, pl.ds(j * _LANES, _LANES)]
                )

        @pl.loop(0, _NCHUNK, step=2)
        def _chunks(k):
            for slot in range(2):
                kk = k + slot
                pltpu.make_async_copy(in_src(kk), inbufs[slot], insem).wait()

                @pl.when(kk >= 2)
                def _():
                    pltpu.make_async_copy(
                        outbufs[slot], out_dst(kk - 2), osems[slot]
                    ).wait()

                compute(slot)

                @pl.when(kk + 2 < _NCHUNK)
                def _():
                    pltpu.async_copy(in_src(kk + 2), inbufs[slot], insem)

                pltpu.async_copy(outbufs[slot], out_dst(kk), osems[slot])

        pltpu.make_async_copy(outbufs[0], out_dst(_NCHUNK - 2), osems[0]).wait()
        pltpu.make_async_copy(outbufs[1], out_dst(_NCHUNK - 1), osems[1]).wait()

    return sc_kernel


_SC_CALL = _make_sc_call()


@jax.jit
def kernel(input_tensor, colors):
    return _SC_CALL(input_tensor, colors)
